# baseline (speedup 1.0000x reference)
import jax
import jax.numpy as jnp
from jax.experimental import pallas as pl

_BT = 512


def _gate_kernel(xa_ref, xb_ref, w_ref, o_ref):
    w = w_ref[...]
    dims = (((1,), (1,)), ((), ()))
    o_ref[0] = jax.lax.dot_general(
        xa_ref[...], w, dimension_numbers=dims,
        preferred_element_type=jnp.float32)
    o_ref[1] = jax.lax.dot_general(
        xb_ref[...], w, dimension_numbers=dims,
        preferred_element_type=jnp.float32)


def kernel(x, wg_weight):
    tokens, model_dim = x.shape
    num_experts = wg_weight.shape[0]
    half_blocks = tokens // (2 * _BT)
    out3 = pl.pallas_call(
        _gate_kernel,
        grid=(half_blocks,),
        in_specs=[
            pl.BlockSpec((_BT, model_dim), lambda i: (i, 0)),
            pl.BlockSpec((_BT, model_dim),
                         lambda i: (i + half_blocks, 0)),
            pl.BlockSpec((num_experts, model_dim), lambda i: (0, 0)),
        ],
        out_specs=pl.BlockSpec((2, _BT, num_experts), lambda i: (0, i, 0)),
        out_shape=jax.ShapeDtypeStruct(
            (2, tokens // 2, num_experts), jnp.float32),
    )(x, x, wg_weight)
    return out3.reshape(tokens, num_experts)


# speedup vs baseline: 1.0724x; 1.0724x over previous
import jax
import jax.numpy as jnp
from jax.experimental import pallas as pl

_BT = 512


def _gate_kernel(x_ref, w_ref, o_ref):
    o_ref[...] = jax.lax.dot_general(
        x_ref[...], w_ref[...],
        dimension_numbers=(((1,), (1,)), ((), ())),
        preferred_element_type=jnp.float32,
    )


def kernel(x, wg_weight):
    tokens, model_dim = x.shape
    num_experts = wg_weight.shape[0]
    return pl.pallas_call(
        _gate_kernel,
        grid=(tokens // _BT,),
        in_specs=[
            pl.BlockSpec((_BT, model_dim), lambda i: (i, 0)),
            pl.BlockSpec((num_experts, model_dim), lambda i: (0, 0)),
        ],
        out_specs=pl.BlockSpec((_BT, num_experts), lambda i: (i, 0)),
        out_shape=jax.ShapeDtypeStruct((tokens, num_experts), jnp.float32),
    )(x, wg_weight)
